# baseline (speedup 1.0000x reference)
import jax
import jax.numpy as jnp
from jax import lax
from jax.experimental import pallas as pl
from jax.experimental.pallas import tpu as pltpu
from jax.experimental.pallas import tpu_sc as plsc

N = 10000
E = 320000
D = 128
NHID = 32
LAT = 16

NC = 2
NS = 16
NW = NC * NS
K = 125
CHUNKS = E // (NW * K)
NP = 10240
ROWS_PER_TILE = NP // NS
DEGW = 16

_sc_mesh = plsc.VectorSubcoreMesh(core_axis_name="c", subcore_axis_name="s")


def _make_deg_kernel():
    def body(e_hbm, ones_hbm, zeros_hbm, out_hbm, acc, dst_v, ones_v, stage_v, sem):
        cid = lax.axis_index("c")
        sid = lax.axis_index("s")
        wid = sid * NC + cid
        row0 = sid * ROWS_PER_TILE
        pltpu.sync_copy(zeros_hbm.at[pl.ds(row0, ROWS_PER_TILE)], stage_v)
        pltpu.sync_copy(stage_v, acc.at[pl.ds(row0, ROWS_PER_TILE)])
        pltpu.sync_copy(ones_hbm, ones_v)
        pltpu.sync_copy(e_hbm.at[1, pl.ds(wid * CHUNKS, CHUNKS)], dst_v)
        plsc.subcore_barrier()

        def step(j, carry):
            pltpu.async_copy(ones_v, acc.at[dst_v.at[j]], sem, add=True)
            return carry

        lax.fori_loop(0, CHUNKS, step, 0)

        def drain(j, carry):
            pltpu.make_async_copy(ones_v, acc.at[dst_v.at[j]], sem).wait()
            return carry

        lax.fori_loop(0, CHUNKS, drain, 0)
        plsc.subcore_barrier()
        pltpu.sync_copy(acc.at[pl.ds(row0, ROWS_PER_TILE)], stage_v)
        pltpu.sync_copy(stage_v, out_hbm.at[cid, pl.ds(row0, ROWS_PER_TILE)])

    return pl.kernel(
        body,
        out_type=jax.ShapeDtypeStruct((NC, NP, DEGW), jnp.float32),
        mesh=_sc_mesh,
        scratch_types=[
            pltpu.VMEM_SHARED((NP, DEGW), jnp.float32),
            pltpu.VMEM((CHUNKS, K), jnp.int32),
            pltpu.VMEM((K, DEGW), jnp.float32),
            pltpu.VMEM((ROWS_PER_TILE, DEGW), jnp.float32),
            pltpu.SemaphoreType.DMA,
        ],
        compiler_params=pltpu.CompilerParams(use_tc_tiling_on_sc=False),
    )


def _make_scatter_kernel(F):
    G = 8

    def body(g_hbm, e_hbm, zeros_hbm, out_hbm, acc,
             src_v, dst_v, bufs, stage_v, gsem0, gsem1, ssem0, ssem1):
        cid = lax.axis_index("c")
        sid = lax.axis_index("s")
        wid = sid * NC + cid
        row0 = sid * ROWS_PER_TILE
        pltpu.sync_copy(zeros_hbm.at[pl.ds(row0, ROWS_PER_TILE)], stage_v)
        pltpu.sync_copy(stage_v, acc.at[pl.ds(row0, ROWS_PER_TILE)])
        pltpu.sync_copy(e_hbm.at[0, pl.ds(wid * CHUNKS, CHUNKS)], src_v)
        pltpu.sync_copy(e_hbm.at[1, pl.ds(wid * CHUNKS, CHUNKS)], dst_v)
        plsc.subcore_barrier()

        def g_wait(j, buf, sem):
            pltpu.make_async_copy(g_hbm.at[src_v.at[j]], buf, sem).wait()

        def s_wait(j, buf, sem):
            pltpu.make_async_copy(buf, acc.at[dst_v.at[j]], sem).wait()

        for b in range(G):
            pltpu.async_copy(g_hbm.at[src_v.at[b]], bufs.at[b], gsem0)

        def step(s, carry):
            jA = 2 * G * s
            jB = jA + G
            @pl.when(s >= 1)
            def _():
                for b in range(G):
                    s_wait(jB - 2 * G + b, bufs.at[G + b], ssem1)
            for b in range(G):
                pltpu.async_copy(g_hbm.at[src_v.at[jB + b]], bufs.at[G + b], gsem1)
            for b in range(G):
                g_wait(jA + b, bufs.at[b], gsem0)
            for b in range(G):
                pltpu.async_copy(bufs.at[b], acc.at[dst_v.at[jA + b]], ssem0, add=True)
            for b in range(G):
                s_wait(jA + b, bufs.at[b], ssem0)
            for b in range(G):
                jn = jnp.minimum(jA + 2 * G + b, CHUNKS - 1)
                pltpu.async_copy(g_hbm.at[src_v.at[jn]], bufs.at[b], gsem0)
            for b in range(G):
                g_wait(jB + b, bufs.at[G + b], gsem1)
            for b in range(G):
                pltpu.async_copy(bufs.at[G + b], acc.at[dst_v.at[jB + b]], ssem1, add=True)
            return carry

        nsteps = CHUNKS // (2 * G)
        lax.fori_loop(0, nsteps, step, 0)
        for b in range(G):
            s_wait(CHUNKS - G + b, bufs.at[G + b], ssem1)
        for b in range(G):
            g_wait(CHUNKS - 1, bufs.at[b], gsem0)
        plsc.subcore_barrier()
        pltpu.sync_copy(acc.at[pl.ds(row0, ROWS_PER_TILE)], stage_v)
        pltpu.sync_copy(stage_v, out_hbm.at[cid, pl.ds(row0, ROWS_PER_TILE)])

    return pl.kernel(
        body,
        out_type=jax.ShapeDtypeStruct((NC, NP, F), jnp.float32),
        mesh=_sc_mesh,
        scratch_types=[
            pltpu.VMEM_SHARED((NP, F), jnp.float32),
            pltpu.VMEM((CHUNKS, K), jnp.int32),
            pltpu.VMEM((CHUNKS, K), jnp.int32),
            pltpu.VMEM((2 * G, K, F), jnp.float32),
            pltpu.VMEM((ROWS_PER_TILE, F), jnp.float32),
            pltpu.SemaphoreType.DMA,
            pltpu.SemaphoreType.DMA,
            pltpu.SemaphoreType.DMA,
            pltpu.SemaphoreType.DMA,
        ],
        compiler_params=pltpu.CompilerParams(use_tc_tiling_on_sc=False),
    )


_RB = 2000


def _prep1_body(dp_ref, x_ref, w1_ref, g1_ref, dinv_ref):
    deg = dp_ref[0, :, :1] + dp_ref[1, :, :1] + 1.0
    dinv = lax.rsqrt(jnp.maximum(deg, 1.0))
    h = jnp.dot(x_ref[...], w1_ref[...], preferred_element_type=jnp.float32)
    g1_ref[...] = h * dinv
    dinv_ref[...] = dinv


def _prep2_body(pp_ref, g_ref, dinv_ref, w_ref, b_ref, out_ref):
    dinv = dinv_ref[...]
    o1 = dinv * (pp_ref[0] + pp_ref[1] + g_ref[...]) + b_ref[...]
    h2 = jnp.dot(o1, w_ref[...], preferred_element_type=jnp.float32)
    out_ref[...] = h2 * dinv


def _zfin_body(qq_ref, g_ref, dinv_ref, b_ref, z_ref):
    z_ref[...] = dinv_ref[...] * (qq_ref[0] + qq_ref[1] + g_ref[...]) + b_ref[...]


_TM = 1024
_TN = 5120


def _adj_body(zr_ref, zc_ref, out_ref):
    prod = lax.dot_general(zr_ref[...], zc_ref[...],
                           (((1,), (1,)), ((), ())),
                           preferred_element_type=jnp.float32)
    out_ref[...] = 0.5 * jnp.tanh(0.5 * prod) + 0.5


def kernel(x, edge_index, W1, b1, W2, b2):
    e3d = edge_index.reshape(2, E // K, K)
    ones_deg = jnp.ones((K, DEGW), jnp.float32)
    zeros_deg = jnp.zeros((NP, DEGW), jnp.float32)
    zeros_h = jnp.zeros((NP, NHID), jnp.float32)
    zeros_l = jnp.zeros((NP, LAT), jnp.float32)
    b1r = b1.reshape(1, NHID)
    b2r = b2.reshape(1, LAT)

    deg_parts = _make_deg_kernel()(e3d, ones_deg, zeros_deg)

    nrb = N // _RB
    g1, dinv = pl.pallas_call(
        _prep1_body,
        grid=(nrb,),
        in_specs=[
            pl.BlockSpec((NC, _RB, DEGW), lambda i: (0, i, 0)),
            pl.BlockSpec((_RB, D), lambda i: (i, 0)),
            pl.BlockSpec((D, NHID), lambda i: (0, 0)),
        ],
        out_specs=[
            pl.BlockSpec((_RB, NHID), lambda i: (i, 0)),
            pl.BlockSpec((_RB, 1), lambda i: (i, 0)),
        ],
        out_shape=[
            jax.ShapeDtypeStruct((N, NHID), jnp.float32),
            jax.ShapeDtypeStruct((N, 1), jnp.float32),
        ],
    )(deg_parts, x, W1)

    parts1 = _make_scatter_kernel(NHID)(g1, e3d, zeros_h)

    g2 = pl.pallas_call(
        _prep2_body,
        grid=(nrb,),
        in_specs=[
            pl.BlockSpec((NC, _RB, NHID), lambda i: (0, i, 0)),
            pl.BlockSpec((_RB, NHID), lambda i: (i, 0)),
            pl.BlockSpec((_RB, 1), lambda i: (i, 0)),
            pl.BlockSpec((NHID, LAT), lambda i: (0, 0)),
            pl.BlockSpec((1, NHID), lambda i: (0, 0)),
        ],
        out_specs=pl.BlockSpec((_RB, LAT), lambda i: (i, 0)),
        out_shape=jax.ShapeDtypeStruct((N, LAT), jnp.float32),
    )(parts1, g1, dinv, W2, b1r)

    parts2 = _make_scatter_kernel(LAT)(g2, e3d, zeros_l)

    z = pl.pallas_call(
        _zfin_body,
        grid=(nrb,),
        in_specs=[
            pl.BlockSpec((NC, _RB, LAT), lambda i: (0, i, 0)),
            pl.BlockSpec((_RB, LAT), lambda i: (i, 0)),
            pl.BlockSpec((_RB, 1), lambda i: (i, 0)),
            pl.BlockSpec((1, LAT), lambda i: (0, 0)),
        ],
        out_specs=pl.BlockSpec((_RB, LAT), lambda i: (i, 0)),
        out_shape=jax.ShapeDtypeStruct((N, LAT), jnp.float32),
    )(parts2, g2, dinv, b2r)

    adj = pl.pallas_call(
        _adj_body,
        grid=(pl.cdiv(N, _TM), pl.cdiv(N, _TN)),
        in_specs=[
            pl.BlockSpec((_TM, LAT), lambda i, j: (i, 0)),
            pl.BlockSpec((_TN, LAT), lambda i, j: (j, 0)),
        ],
        out_specs=pl.BlockSpec((_TM, _TN), lambda i, j: (i, j)),
        out_shape=jax.ShapeDtypeStruct((N, N), jnp.float32),
        compiler_params=pltpu.CompilerParams(
            dimension_semantics=("parallel", "parallel")),
    )(z, z)

    return adj, z


# speedup vs baseline: 1.0267x; 1.0267x over previous
import jax
import jax.numpy as jnp
from jax import lax
from jax.experimental import pallas as pl
from jax.experimental.pallas import tpu as pltpu
from jax.experimental.pallas import tpu_sc as plsc

N = 10000
E = 320000
D = 128
NHID = 32
LAT = 16

NC = 2
NS = 16
NW = NC * NS
K = 125
CHUNKS = E // (NW * K)
NP = 10240
ROWS_PER_TILE = NP // NS
DEGW = 16

_sc_mesh = plsc.VectorSubcoreMesh(core_axis_name="c", subcore_axis_name="s")


def _make_deg_kernel():
    def body(e_hbm, ones_hbm, zeros_hbm, out_hbm, acc, dst_v, ones_v, stage_v, sem):
        cid = lax.axis_index("c")
        sid = lax.axis_index("s")
        wid = sid * NC + cid
        row0 = sid * ROWS_PER_TILE
        pltpu.sync_copy(zeros_hbm.at[pl.ds(row0, ROWS_PER_TILE)], stage_v)
        pltpu.sync_copy(stage_v, acc.at[pl.ds(row0, ROWS_PER_TILE)])
        pltpu.sync_copy(ones_hbm, ones_v)
        pltpu.sync_copy(e_hbm.at[1, pl.ds(wid * CHUNKS, CHUNKS)], dst_v)
        plsc.subcore_barrier()

        def step(j, carry):
            pltpu.async_copy(ones_v, acc.at[dst_v.at[j]], sem, add=True)
            return carry

        lax.fori_loop(0, CHUNKS, step, 0)

        def drain(j, carry):
            pltpu.make_async_copy(ones_v, acc.at[dst_v.at[j]], sem).wait()
            return carry

        lax.fori_loop(0, CHUNKS, drain, 0)
        plsc.subcore_barrier()
        pltpu.sync_copy(acc.at[pl.ds(row0, ROWS_PER_TILE)], stage_v)
        pltpu.sync_copy(stage_v, out_hbm.at[cid, pl.ds(row0, ROWS_PER_TILE)])

    return pl.kernel(
        body,
        out_type=jax.ShapeDtypeStruct((NC, NP, DEGW), jnp.float32),
        mesh=_sc_mesh,
        scratch_types=[
            pltpu.VMEM_SHARED((NP, DEGW), jnp.float32),
            pltpu.VMEM((CHUNKS, K), jnp.int32),
            pltpu.VMEM((K, DEGW), jnp.float32),
            pltpu.VMEM((ROWS_PER_TILE, DEGW), jnp.float32),
            pltpu.SemaphoreType.DMA,
        ],
        compiler_params=pltpu.CompilerParams(use_tc_tiling_on_sc=False),
    )


def _make_scatter_kernel(F):
    G = 8

    def body(g_hbm, e_hbm, zeros_hbm, out_hbm, acc,
             src_v, dst_v, bufs, stage_v, gsem0, gsem1, ssem0, ssem1):
        cid = lax.axis_index("c")
        sid = lax.axis_index("s")
        wid = sid * NC + cid
        row0 = sid * ROWS_PER_TILE
        pltpu.sync_copy(zeros_hbm.at[pl.ds(row0, ROWS_PER_TILE)], stage_v)
        pltpu.sync_copy(stage_v, acc.at[pl.ds(row0, ROWS_PER_TILE)])
        pltpu.sync_copy(e_hbm.at[0, pl.ds(wid * CHUNKS, CHUNKS)], src_v)
        pltpu.sync_copy(e_hbm.at[1, pl.ds(wid * CHUNKS, CHUNKS)], dst_v)
        plsc.subcore_barrier()

        def g_wait(j, buf, sem):
            pltpu.make_async_copy(g_hbm.at[src_v.at[j]], buf, sem).wait()

        def s_wait(j, buf, sem):
            pltpu.make_async_copy(buf, acc.at[dst_v.at[j]], sem).wait()

        for b in range(G):
            pltpu.async_copy(g_hbm.at[src_v.at[b]], bufs.at[b], gsem0)

        def step(s, carry):
            jA = 2 * G * s
            jB = jA + G
            @pl.when(s >= 1)
            def _():
                for b in range(G):
                    s_wait(jB - 2 * G + b, bufs.at[G + b], ssem1)
            for b in range(G):
                pltpu.async_copy(g_hbm.at[src_v.at[jB + b]], bufs.at[G + b], gsem1)
            for b in range(G):
                g_wait(jA + b, bufs.at[b], gsem0)
            for b in range(G):
                pltpu.async_copy(bufs.at[b], acc.at[dst_v.at[jA + b]], ssem0, add=True)
            for b in range(G):
                s_wait(jA + b, bufs.at[b], ssem0)
            for b in range(G):
                jn = jnp.minimum(jA + 2 * G + b, CHUNKS - 1)
                pltpu.async_copy(g_hbm.at[src_v.at[jn]], bufs.at[b], gsem0)
            for b in range(G):
                g_wait(jB + b, bufs.at[G + b], gsem1)
            for b in range(G):
                pltpu.async_copy(bufs.at[G + b], acc.at[dst_v.at[jB + b]], ssem1, add=True)
            return carry

        nsteps = CHUNKS // (2 * G)
        lax.fori_loop(0, nsteps, step, 0)
        for b in range(G):
            s_wait(CHUNKS - G + b, bufs.at[G + b], ssem1)
        for b in range(G):
            g_wait(CHUNKS - 1, bufs.at[b], gsem0)
        plsc.subcore_barrier()
        pltpu.sync_copy(acc.at[pl.ds(row0, ROWS_PER_TILE)], stage_v)
        pltpu.sync_copy(stage_v, out_hbm.at[cid, pl.ds(row0, ROWS_PER_TILE)])

    return pl.kernel(
        body,
        out_type=jax.ShapeDtypeStruct((NC, NP, F), jnp.float32),
        mesh=_sc_mesh,
        scratch_types=[
            pltpu.VMEM_SHARED((NP, F), jnp.float32),
            pltpu.VMEM((CHUNKS, K), jnp.int32),
            pltpu.VMEM((CHUNKS, K), jnp.int32),
            pltpu.VMEM((2 * G, K, F), jnp.float32),
            pltpu.VMEM((ROWS_PER_TILE, F), jnp.float32),
            pltpu.SemaphoreType.DMA,
            pltpu.SemaphoreType.DMA,
            pltpu.SemaphoreType.DMA,
            pltpu.SemaphoreType.DMA,
        ],
        compiler_params=pltpu.CompilerParams(use_tc_tiling_on_sc=False),
    )


_RB = 2000


def _prep1_body(dp_ref, x_ref, w1_ref, g1_ref, dinv_ref):
    deg = dp_ref[0, :, :1] + dp_ref[1, :, :1] + 1.0
    dinv = lax.rsqrt(jnp.maximum(deg, 1.0))
    h = jnp.dot(x_ref[...], w1_ref[...], preferred_element_type=jnp.float32)
    g1_ref[...] = h * dinv
    dinv_ref[...] = dinv


def _prep2_body(pp_ref, g_ref, dinv_ref, w_ref, b_ref, out_ref):
    dinv = dinv_ref[...]
    o1 = dinv * (pp_ref[0] + pp_ref[1] + g_ref[...]) + b_ref[...]
    h2 = jnp.dot(o1, w_ref[...], preferred_element_type=jnp.float32)
    out_ref[...] = h2 * dinv


def _zfin_body(qq_ref, g_ref, dinv_ref, b_ref, z_ref):
    z_ref[...] = dinv_ref[...] * (qq_ref[0] + qq_ref[1] + g_ref[...]) + b_ref[...]


_TM = 2048
_TN = 2560


def _adj_body(zr_ref, zc_ref, out_ref):
    prod = lax.dot_general(zr_ref[...], zc_ref[...],
                           (((1,), (1,)), ((), ())),
                           preferred_element_type=jnp.float32)
    out_ref[...] = 0.5 * jnp.tanh(0.5 * prod) + 0.5


def kernel(x, edge_index, W1, b1, W2, b2):
    e3d = edge_index.reshape(2, E // K, K)
    ones_deg = jnp.ones((K, DEGW), jnp.float32)
    zeros_deg = jnp.zeros((NP, DEGW), jnp.float32)
    zeros_h = jnp.zeros((NP, NHID), jnp.float32)
    zeros_l = jnp.zeros((NP, LAT), jnp.float32)
    b1r = b1.reshape(1, NHID)
    b2r = b2.reshape(1, LAT)

    deg_parts = _make_deg_kernel()(e3d, ones_deg, zeros_deg)

    nrb = N // _RB
    g1, dinv = pl.pallas_call(
        _prep1_body,
        grid=(nrb,),
        in_specs=[
            pl.BlockSpec((NC, _RB, DEGW), lambda i: (0, i, 0)),
            pl.BlockSpec((_RB, D), lambda i: (i, 0)),
            pl.BlockSpec((D, NHID), lambda i: (0, 0)),
        ],
        out_specs=[
            pl.BlockSpec((_RB, NHID), lambda i: (i, 0)),
            pl.BlockSpec((_RB, 1), lambda i: (i, 0)),
        ],
        out_shape=[
            jax.ShapeDtypeStruct((N, NHID), jnp.float32),
            jax.ShapeDtypeStruct((N, 1), jnp.float32),
        ],
    )(deg_parts, x, W1)

    parts1 = _make_scatter_kernel(NHID)(g1, e3d, zeros_h)

    g2 = pl.pallas_call(
        _prep2_body,
        grid=(nrb,),
        in_specs=[
            pl.BlockSpec((NC, _RB, NHID), lambda i: (0, i, 0)),
            pl.BlockSpec((_RB, NHID), lambda i: (i, 0)),
            pl.BlockSpec((_RB, 1), lambda i: (i, 0)),
            pl.BlockSpec((NHID, LAT), lambda i: (0, 0)),
            pl.BlockSpec((1, NHID), lambda i: (0, 0)),
        ],
        out_specs=pl.BlockSpec((_RB, LAT), lambda i: (i, 0)),
        out_shape=jax.ShapeDtypeStruct((N, LAT), jnp.float32),
    )(parts1, g1, dinv, W2, b1r)

    parts2 = _make_scatter_kernel(LAT)(g2, e3d, zeros_l)

    z = pl.pallas_call(
        _zfin_body,
        grid=(nrb,),
        in_specs=[
            pl.BlockSpec((NC, _RB, LAT), lambda i: (0, i, 0)),
            pl.BlockSpec((_RB, LAT), lambda i: (i, 0)),
            pl.BlockSpec((_RB, 1), lambda i: (i, 0)),
            pl.BlockSpec((1, LAT), lambda i: (0, 0)),
        ],
        out_specs=pl.BlockSpec((_RB, LAT), lambda i: (i, 0)),
        out_shape=jax.ShapeDtypeStruct((N, LAT), jnp.float32),
    )(parts2, g2, dinv, b2r)

    adj = pl.pallas_call(
        _adj_body,
        grid=(pl.cdiv(N, _TM), pl.cdiv(N, _TN)),
        in_specs=[
            pl.BlockSpec((_TM, LAT), lambda i, j: (i, 0)),
            pl.BlockSpec((_TN, LAT), lambda i, j: (j, 0)),
        ],
        out_specs=pl.BlockSpec((_TM, _TN), lambda i, j: (i, j)),
        out_shape=jax.ShapeDtypeStruct((N, N), jnp.float32),
        compiler_params=pltpu.CompilerParams(
            dimension_semantics=("parallel", "parallel")),
    )(z, z)

    return adj, z


# speedup vs baseline: 1.0324x; 1.0055x over previous
import jax
import jax.numpy as jnp
from jax import lax
from jax.experimental import pallas as pl
from jax.experimental.pallas import tpu as pltpu
from jax.experimental.pallas import tpu_sc as plsc

N = 10000
E = 320000
D = 128
NHID = 32
LAT = 16

NC = 2
NS = 16
NW = NC * NS
K = 125
CHUNKS = E // (NW * K)
NP = 10240
ROWS_PER_TILE = NP // NS
DEGW = 16

_sc_mesh = plsc.VectorSubcoreMesh(core_axis_name="c", subcore_axis_name="s")


def _make_deg_kernel():
    def body(e_hbm, ones_hbm, zeros_hbm, out_hbm, acc, dst_v, ones_v, stage_v, sem):
        cid = lax.axis_index("c")
        sid = lax.axis_index("s")
        wid = sid * NC + cid
        row0 = sid * ROWS_PER_TILE
        pltpu.sync_copy(zeros_hbm.at[pl.ds(row0, ROWS_PER_TILE)], stage_v)
        pltpu.sync_copy(stage_v, acc.at[pl.ds(row0, ROWS_PER_TILE)])
        pltpu.sync_copy(ones_hbm, ones_v)
        pltpu.sync_copy(e_hbm.at[1, pl.ds(wid * CHUNKS, CHUNKS)], dst_v)
        plsc.subcore_barrier()

        def step(j, carry):
            pltpu.async_copy(ones_v, acc.at[dst_v.at[j]], sem, add=True)
            return carry

        lax.fori_loop(0, CHUNKS, step, 0)

        def drain(j, carry):
            pltpu.make_async_copy(ones_v, acc.at[dst_v.at[j]], sem).wait()
            return carry

        lax.fori_loop(0, CHUNKS, drain, 0)
        plsc.subcore_barrier()
        pltpu.sync_copy(acc.at[pl.ds(row0, ROWS_PER_TILE)], stage_v)
        pltpu.sync_copy(stage_v, out_hbm.at[cid, pl.ds(row0, ROWS_PER_TILE)])

    return pl.kernel(
        body,
        out_type=jax.ShapeDtypeStruct((NC, NP, DEGW), jnp.float32),
        mesh=_sc_mesh,
        scratch_types=[
            pltpu.VMEM_SHARED((NP, DEGW), jnp.float32),
            pltpu.VMEM((CHUNKS, K), jnp.int32),
            pltpu.VMEM((K, DEGW), jnp.float32),
            pltpu.VMEM((ROWS_PER_TILE, DEGW), jnp.float32),
            pltpu.SemaphoreType.DMA,
        ],
        compiler_params=pltpu.CompilerParams(use_tc_tiling_on_sc=False),
    )


def _make_scatter_kernel(F):
    G = 8

    def body(g_hbm, e_hbm, zeros_hbm, out_hbm, acc,
             src_v, dst_v, bufs, stage_v, gsem0, gsem1, ssem0, ssem1):
        cid = lax.axis_index("c")
        sid = lax.axis_index("s")
        wid = sid * NC + cid
        row0 = sid * ROWS_PER_TILE
        pltpu.sync_copy(zeros_hbm.at[pl.ds(row0, ROWS_PER_TILE)], stage_v)
        pltpu.sync_copy(stage_v, acc.at[pl.ds(row0, ROWS_PER_TILE)])
        pltpu.sync_copy(e_hbm.at[0, pl.ds(wid * CHUNKS, CHUNKS)], src_v)
        pltpu.sync_copy(e_hbm.at[1, pl.ds(wid * CHUNKS, CHUNKS)], dst_v)
        plsc.subcore_barrier()

        def g_wait(j, buf, sem):
            pltpu.make_async_copy(g_hbm.at[src_v.at[j]], buf, sem).wait()

        def s_wait(j, buf, sem):
            pltpu.make_async_copy(buf, acc.at[dst_v.at[j]], sem).wait()

        for b in range(G):
            pltpu.async_copy(g_hbm.at[src_v.at[b]], bufs.at[b], gsem0)

        def step(s, carry):
            jA = 2 * G * s
            jB = jA + G
            @pl.when(s >= 1)
            def _():
                for b in range(G):
                    s_wait(jB - 2 * G + b, bufs.at[G + b], ssem1)
            for b in range(G):
                pltpu.async_copy(g_hbm.at[src_v.at[jB + b]], bufs.at[G + b], gsem1)
            for b in range(G):
                g_wait(jA + b, bufs.at[b], gsem0)
            for b in range(G):
                pltpu.async_copy(bufs.at[b], acc.at[dst_v.at[jA + b]], ssem0, add=True)
            for b in range(G):
                s_wait(jA + b, bufs.at[b], ssem0)
            for b in range(G):
                jn = jnp.minimum(jA + 2 * G + b, CHUNKS - 1)
                pltpu.async_copy(g_hbm.at[src_v.at[jn]], bufs.at[b], gsem0)
            for b in range(G):
                g_wait(jB + b, bufs.at[G + b], gsem1)
            for b in range(G):
                pltpu.async_copy(bufs.at[G + b], acc.at[dst_v.at[jB + b]], ssem1, add=True)
            return carry

        nsteps = CHUNKS // (2 * G)
        lax.fori_loop(0, nsteps, step, 0)
        for b in range(G):
            s_wait(CHUNKS - G + b, bufs.at[G + b], ssem1)
        for b in range(G):
            g_wait(CHUNKS - 1, bufs.at[b], gsem0)
        plsc.subcore_barrier()
        pltpu.sync_copy(acc.at[pl.ds(row0, ROWS_PER_TILE)], stage_v)
        pltpu.sync_copy(stage_v, out_hbm.at[cid, pl.ds(row0, ROWS_PER_TILE)])

    return pl.kernel(
        body,
        out_type=jax.ShapeDtypeStruct((NC, NP, F), jnp.float32),
        mesh=_sc_mesh,
        scratch_types=[
            pltpu.VMEM_SHARED((NP, F), jnp.float32),
            pltpu.VMEM((CHUNKS, K), jnp.int32),
            pltpu.VMEM((CHUNKS, K), jnp.int32),
            pltpu.VMEM((2 * G, K, F), jnp.float32),
            pltpu.VMEM((ROWS_PER_TILE, F), jnp.float32),
            pltpu.SemaphoreType.DMA,
            pltpu.SemaphoreType.DMA,
            pltpu.SemaphoreType.DMA,
            pltpu.SemaphoreType.DMA,
        ],
        compiler_params=pltpu.CompilerParams(use_tc_tiling_on_sc=False),
    )


_RB = 2000


def _mm1_body(x_ref, w1_ref, h_ref):
    h_ref[...] = jnp.dot(x_ref[...], w1_ref[...],
                         preferred_element_type=jnp.float32)


def _prep1_body(dp_ref, h_ref, g1_ref, dinv_ref):
    deg = dp_ref[0, :, :1] + dp_ref[1, :, :1] + 1.0
    dinv = lax.rsqrt(jnp.maximum(deg, 1.0))
    g1_ref[...] = h_ref[...] * dinv
    dinv_ref[...] = dinv


def _prep2_body(pp_ref, g_ref, dinv_ref, w_ref, b_ref, out_ref):
    dinv = dinv_ref[...]
    o1 = dinv * (pp_ref[0] + pp_ref[1] + g_ref[...]) + b_ref[...]
    h2 = jnp.dot(o1, w_ref[...], preferred_element_type=jnp.float32)
    out_ref[...] = h2 * dinv


def _zfin_body(qq_ref, g_ref, dinv_ref, b_ref, z_ref):
    z_ref[...] = dinv_ref[...] * (qq_ref[0] + qq_ref[1] + g_ref[...]) + b_ref[...]


_TM = 2048
_TN = 2560


def _adj_body(zr_ref, zc_ref, out_ref):
    prod = lax.dot_general(zr_ref[...], zc_ref[...],
                           (((1,), (1,)), ((), ())),
                           preferred_element_type=jnp.float32)
    out_ref[...] = 0.5 * jnp.tanh(0.5 * prod) + 0.5


def kernel(x, edge_index, W1, b1, W2, b2):
    e3d = edge_index.reshape(2, E // K, K)
    ones_deg = jnp.ones((K, DEGW), jnp.float32)
    zeros_deg = jnp.zeros((NP, DEGW), jnp.float32)
    zeros_h = jnp.zeros((NP, NHID), jnp.float32)
    zeros_l = jnp.zeros((NP, LAT), jnp.float32)
    b1r = b1.reshape(1, NHID)
    b2r = b2.reshape(1, LAT)

    deg_parts = _make_deg_kernel()(e3d, ones_deg, zeros_deg)

    nrb = N // _RB
    h1 = pl.pallas_call(
        _mm1_body,
        grid=(nrb,),
        in_specs=[
            pl.BlockSpec((_RB, D), lambda i: (i, 0)),
            pl.BlockSpec((D, NHID), lambda i: (0, 0)),
        ],
        out_specs=pl.BlockSpec((_RB, NHID), lambda i: (i, 0)),
        out_shape=jax.ShapeDtypeStruct((N, NHID), jnp.float32),
    )(x, W1)

    g1, dinv = pl.pallas_call(
        _prep1_body,
        grid=(nrb,),
        in_specs=[
            pl.BlockSpec((NC, _RB, DEGW), lambda i: (0, i, 0)),
            pl.BlockSpec((_RB, NHID), lambda i: (i, 0)),
        ],
        out_specs=[
            pl.BlockSpec((_RB, NHID), lambda i: (i, 0)),
            pl.BlockSpec((_RB, 1), lambda i: (i, 0)),
        ],
        out_shape=[
            jax.ShapeDtypeStruct((N, NHID), jnp.float32),
            jax.ShapeDtypeStruct((N, 1), jnp.float32),
        ],
    )(deg_parts, h1)

    parts1 = _make_scatter_kernel(NHID)(g1, e3d, zeros_h)

    g2 = pl.pallas_call(
        _prep2_body,
        grid=(nrb,),
        in_specs=[
            pl.BlockSpec((NC, _RB, NHID), lambda i: (0, i, 0)),
            pl.BlockSpec((_RB, NHID), lambda i: (i, 0)),
            pl.BlockSpec((_RB, 1), lambda i: (i, 0)),
            pl.BlockSpec((NHID, LAT), lambda i: (0, 0)),
            pl.BlockSpec((1, NHID), lambda i: (0, 0)),
        ],
        out_specs=pl.BlockSpec((_RB, LAT), lambda i: (i, 0)),
        out_shape=jax.ShapeDtypeStruct((N, LAT), jnp.float32),
    )(parts1, g1, dinv, W2, b1r)

    parts2 = _make_scatter_kernel(LAT)(g2, e3d, zeros_l)

    z = pl.pallas_call(
        _zfin_body,
        grid=(nrb,),
        in_specs=[
            pl.BlockSpec((NC, _RB, LAT), lambda i: (0, i, 0)),
            pl.BlockSpec((_RB, LAT), lambda i: (i, 0)),
            pl.BlockSpec((_RB, 1), lambda i: (i, 0)),
            pl.BlockSpec((1, LAT), lambda i: (0, 0)),
        ],
        out_specs=pl.BlockSpec((_RB, LAT), lambda i: (i, 0)),
        out_shape=jax.ShapeDtypeStruct((N, LAT), jnp.float32),
    )(parts2, g2, dinv, b2r)

    adj = pl.pallas_call(
        _adj_body,
        grid=(pl.cdiv(N, _TM), pl.cdiv(N, _TN)),
        in_specs=[
            pl.BlockSpec((_TM, LAT), lambda i, j: (i, 0)),
            pl.BlockSpec((_TN, LAT), lambda i, j: (j, 0)),
        ],
        out_specs=pl.BlockSpec((_TM, _TN), lambda i, j: (i, j)),
        out_shape=jax.ShapeDtypeStruct((N, N), jnp.float32),
        compiler_params=pltpu.CompilerParams(
            dimension_semantics=("parallel", "parallel")),
    )(z, z)

    return adj, z


# speedup vs baseline: 1.0379x; 1.0054x over previous
import jax
import jax.numpy as jnp
from jax import lax
from jax.experimental import pallas as pl
from jax.experimental.pallas import tpu as pltpu
from jax.experimental.pallas import tpu_sc as plsc

N = 10000
E = 320000
D = 128
NHID = 32
LAT = 16

NC = 2
NS = 16
NW = NC * NS
K = 125
CHUNKS = E // (NW * K)
NP = 10240
ROWS_PER_TILE = NP // NS
DEGW = 16

_sc_mesh = plsc.VectorSubcoreMesh(core_axis_name="c", subcore_axis_name="s")


def _make_deg_kernel():
    def body(e_hbm, ones_hbm, zeros_hbm, out_hbm, acc, dst_v, ones_v, stage_v, sem):
        cid = lax.axis_index("c")
        sid = lax.axis_index("s")
        wid = sid * NC + cid
        row0 = sid * ROWS_PER_TILE
        pltpu.sync_copy(zeros_hbm.at[pl.ds(row0, ROWS_PER_TILE)], stage_v)
        pltpu.sync_copy(stage_v, acc.at[pl.ds(row0, ROWS_PER_TILE)])
        pltpu.sync_copy(ones_hbm, ones_v)
        pltpu.sync_copy(e_hbm.at[1, pl.ds(wid * CHUNKS, CHUNKS)], dst_v)
        plsc.subcore_barrier()

        def step(j, carry):
            pltpu.async_copy(ones_v, acc.at[dst_v.at[j]], sem, add=True)
            return carry

        lax.fori_loop(0, CHUNKS, step, 0)

        def drain(j, carry):
            pltpu.make_async_copy(ones_v, acc.at[dst_v.at[j]], sem).wait()
            return carry

        lax.fori_loop(0, CHUNKS, drain, 0)
        plsc.subcore_barrier()
        pltpu.sync_copy(acc.at[pl.ds(row0, ROWS_PER_TILE)], stage_v)
        pltpu.sync_copy(stage_v, out_hbm.at[cid, pl.ds(row0, ROWS_PER_TILE)])

    return pl.kernel(
        body,
        out_type=jax.ShapeDtypeStruct((NC, NP, DEGW), jnp.float32),
        mesh=_sc_mesh,
        scratch_types=[
            pltpu.VMEM_SHARED((NP, DEGW), jnp.float32),
            pltpu.VMEM((CHUNKS, K), jnp.int32),
            pltpu.VMEM((K, DEGW), jnp.float32),
            pltpu.VMEM((ROWS_PER_TILE, DEGW), jnp.float32),
            pltpu.SemaphoreType.DMA,
        ],
        compiler_params=pltpu.CompilerParams(use_tc_tiling_on_sc=False),
    )


def _make_scatter_kernel(F, stage_g=False):
    G = 8

    def body(g_hbm, e_hbm, zeros_hbm, out_hbm, acc, spg,
             src_v, dst_v, bufs, stage_v, gsem0, gsem1, ssem0, ssem1):
        gsrc = spg if stage_g else g_hbm
        cid = lax.axis_index("c")
        sid = lax.axis_index("s")
        wid = sid * NC + cid
        row0 = sid * ROWS_PER_TILE
        pltpu.sync_copy(zeros_hbm.at[pl.ds(row0, ROWS_PER_TILE)], stage_v)
        pltpu.sync_copy(stage_v, acc.at[pl.ds(row0, ROWS_PER_TILE)])
        if stage_g:
            grow0 = sid * (N // NS)
            pltpu.sync_copy(g_hbm.at[pl.ds(grow0, N // NS)], stage_v.at[pl.ds(0, N // NS)])
            pltpu.sync_copy(stage_v.at[pl.ds(0, N // NS)], spg.at[pl.ds(grow0, N // NS)])
        pltpu.sync_copy(e_hbm.at[0, pl.ds(wid * CHUNKS, CHUNKS)], src_v)
        pltpu.sync_copy(e_hbm.at[1, pl.ds(wid * CHUNKS, CHUNKS)], dst_v)
        plsc.subcore_barrier()

        def g_wait(j, buf, sem):
            pltpu.make_async_copy(gsrc.at[src_v.at[j]], buf, sem).wait()

        def s_wait(j, buf, sem):
            pltpu.make_async_copy(buf, acc.at[dst_v.at[j]], sem).wait()

        for b in range(G):
            pltpu.async_copy(gsrc.at[src_v.at[b]], bufs.at[b], gsem0)

        def step(s, carry):
            jA = 2 * G * s
            jB = jA + G
            @pl.when(s >= 1)
            def _():
                for b in range(G):
                    s_wait(jB - 2 * G + b, bufs.at[G + b], ssem1)
            for b in range(G):
                pltpu.async_copy(gsrc.at[src_v.at[jB + b]], bufs.at[G + b], gsem1)
            for b in range(G):
                g_wait(jA + b, bufs.at[b], gsem0)
            for b in range(G):
                pltpu.async_copy(bufs.at[b], acc.at[dst_v.at[jA + b]], ssem0, add=True)
            for b in range(G):
                s_wait(jA + b, bufs.at[b], ssem0)
            for b in range(G):
                jn = jnp.minimum(jA + 2 * G + b, CHUNKS - 1)
                pltpu.async_copy(gsrc.at[src_v.at[jn]], bufs.at[b], gsem0)
            for b in range(G):
                g_wait(jB + b, bufs.at[G + b], gsem1)
            for b in range(G):
                pltpu.async_copy(bufs.at[G + b], acc.at[dst_v.at[jB + b]], ssem1, add=True)
            return carry

        nsteps = CHUNKS // (2 * G)
        lax.fori_loop(0, nsteps, step, 0)
        for b in range(G):
            s_wait(CHUNKS - G + b, bufs.at[G + b], ssem1)
        for b in range(G):
            g_wait(CHUNKS - 1, bufs.at[b], gsem0)
        plsc.subcore_barrier()
        pltpu.sync_copy(acc.at[pl.ds(row0, ROWS_PER_TILE)], stage_v)
        pltpu.sync_copy(stage_v, out_hbm.at[cid, pl.ds(row0, ROWS_PER_TILE)])

    return pl.kernel(
        body,
        out_type=jax.ShapeDtypeStruct((NC, NP, F), jnp.float32),
        mesh=_sc_mesh,
        scratch_types=[
            pltpu.VMEM_SHARED((NP, F), jnp.float32),
            pltpu.VMEM_SHARED((N if stage_g else 8, F), jnp.float32),
            pltpu.VMEM((CHUNKS, K), jnp.int32),
            pltpu.VMEM((CHUNKS, K), jnp.int32),
            pltpu.VMEM((2 * G, K, F), jnp.float32),
            pltpu.VMEM((ROWS_PER_TILE, F), jnp.float32),
            pltpu.SemaphoreType.DMA,
            pltpu.SemaphoreType.DMA,
            pltpu.SemaphoreType.DMA,
            pltpu.SemaphoreType.DMA,
        ],
        compiler_params=pltpu.CompilerParams(use_tc_tiling_on_sc=False),
    )


_RB = 2000


def _mm1_body(x_ref, w1_ref, h_ref):
    h_ref[...] = jnp.dot(x_ref[...], w1_ref[...],
                         preferred_element_type=jnp.float32)


def _prep1_body(dp_ref, h_ref, g1_ref, dinv_ref):
    deg = dp_ref[0, :, :1] + dp_ref[1, :, :1] + 1.0
    dinv = lax.rsqrt(jnp.maximum(deg, 1.0))
    g1_ref[...] = h_ref[...] * dinv
    dinv_ref[...] = dinv


def _prep2_body(pp_ref, g_ref, dinv_ref, w_ref, b_ref, out_ref):
    dinv = dinv_ref[...]
    o1 = dinv * (pp_ref[0] + pp_ref[1] + g_ref[...]) + b_ref[...]
    h2 = jnp.dot(o1, w_ref[...], preferred_element_type=jnp.float32)
    out_ref[...] = h2 * dinv


def _zfin_body(qq_ref, g_ref, dinv_ref, b_ref, z_ref):
    z_ref[...] = dinv_ref[...] * (qq_ref[0] + qq_ref[1] + g_ref[...]) + b_ref[...]


_TM = 2048
_TN = 2560


def _adj_body(zr_ref, zc_ref, out_ref):
    prod = lax.dot_general(zr_ref[...], zc_ref[...],
                           (((1,), (1,)), ((), ())),
                           preferred_element_type=jnp.float32)
    out_ref[...] = 0.5 * jnp.tanh(0.5 * prod) + 0.5


def kernel(x, edge_index, W1, b1, W2, b2):
    e3d = edge_index.reshape(2, E // K, K)
    ones_deg = jnp.ones((K, DEGW), jnp.float32)
    zeros_deg = jnp.zeros((NP, DEGW), jnp.float32)
    zeros_h = jnp.zeros((NP, NHID), jnp.float32)
    zeros_l = jnp.zeros((NP, LAT), jnp.float32)
    b1r = b1.reshape(1, NHID)
    b2r = b2.reshape(1, LAT)

    deg_parts = _make_deg_kernel()(e3d, ones_deg, zeros_deg)

    nrb = N // _RB
    h1 = pl.pallas_call(
        _mm1_body,
        grid=(nrb,),
        in_specs=[
            pl.BlockSpec((_RB, D), lambda i: (i, 0)),
            pl.BlockSpec((D, NHID), lambda i: (0, 0)),
        ],
        out_specs=pl.BlockSpec((_RB, NHID), lambda i: (i, 0)),
        out_shape=jax.ShapeDtypeStruct((N, NHID), jnp.float32),
    )(x, W1)

    g1, dinv = pl.pallas_call(
        _prep1_body,
        grid=(nrb,),
        in_specs=[
            pl.BlockSpec((NC, _RB, DEGW), lambda i: (0, i, 0)),
            pl.BlockSpec((_RB, NHID), lambda i: (i, 0)),
        ],
        out_specs=[
            pl.BlockSpec((_RB, NHID), lambda i: (i, 0)),
            pl.BlockSpec((_RB, 1), lambda i: (i, 0)),
        ],
        out_shape=[
            jax.ShapeDtypeStruct((N, NHID), jnp.float32),
            jax.ShapeDtypeStruct((N, 1), jnp.float32),
        ],
    )(deg_parts, h1)

    parts1 = _make_scatter_kernel(NHID)(g1, e3d, zeros_h)

    g2 = pl.pallas_call(
        _prep2_body,
        grid=(nrb,),
        in_specs=[
            pl.BlockSpec((NC, _RB, NHID), lambda i: (0, i, 0)),
            pl.BlockSpec((_RB, NHID), lambda i: (i, 0)),
            pl.BlockSpec((_RB, 1), lambda i: (i, 0)),
            pl.BlockSpec((NHID, LAT), lambda i: (0, 0)),
            pl.BlockSpec((1, NHID), lambda i: (0, 0)),
        ],
        out_specs=pl.BlockSpec((_RB, LAT), lambda i: (i, 0)),
        out_shape=jax.ShapeDtypeStruct((N, LAT), jnp.float32),
    )(parts1, g1, dinv, W2, b1r)

    parts2 = _make_scatter_kernel(LAT, stage_g=True)(g2, e3d, zeros_l)

    z = pl.pallas_call(
        _zfin_body,
        grid=(nrb,),
        in_specs=[
            pl.BlockSpec((NC, _RB, LAT), lambda i: (0, i, 0)),
            pl.BlockSpec((_RB, LAT), lambda i: (i, 0)),
            pl.BlockSpec((_RB, 1), lambda i: (i, 0)),
            pl.BlockSpec((1, LAT), lambda i: (0, 0)),
        ],
        out_specs=pl.BlockSpec((_RB, LAT), lambda i: (i, 0)),
        out_shape=jax.ShapeDtypeStruct((N, LAT), jnp.float32),
    )(parts2, g2, dinv, b2r)

    adj = pl.pallas_call(
        _adj_body,
        grid=(pl.cdiv(N, _TM), pl.cdiv(N, _TN)),
        in_specs=[
            pl.BlockSpec((_TM, LAT), lambda i, j: (i, 0)),
            pl.BlockSpec((_TN, LAT), lambda i, j: (j, 0)),
        ],
        out_specs=pl.BlockSpec((_TM, _TN), lambda i, j: (i, j)),
        out_shape=jax.ShapeDtypeStruct((N, N), jnp.float32),
        compiler_params=pltpu.CompilerParams(
            dimension_semantics=("parallel", "parallel")),
    )(z, z)

    return adj, z
